# baseline (speedup 1.0000x reference)
import functools

import jax
import jax.numpy as jnp
from jax import lax
from jax.experimental import pallas as pl
from jax.experimental.pallas import tpu as pltpu
from jax.experimental.pallas import tpu_sc as plsc

S = 64
PLANE = S * S
NC, NS, L = 2, 16, 16
B = 8
ZPW = 16
N_PTS = S * S * S


@functools.lru_cache(maxsize=None)
def _build():
    mesh = plsc.VectorSubcoreMesh(core_axis_name="c", subcore_axis_name="s",
                                  num_cores=NC, num_subcores=NS)
    return functools.partial(
        pl.kernel,
        out_type=jax.ShapeDtypeStruct((3, B, N_PTS), jnp.float32),
        mesh=mesh,
        compiler_params=pltpu.CompilerParams(needs_layout_passes=False),
        scratch_types=[
            pltpu.VMEM((S + 1, S), jnp.float32),
            pltpu.VMEM((S + 1, S), jnp.float32),
            pltpu.VMEM((S + 1, S), jnp.float32),
            pltpu.VMEM((128 + PLANE,), jnp.float32),
            pltpu.VMEM((128 + PLANE,), jnp.float32),
            pltpu.VMEM((128 + PLANE,), jnp.float32),
            pltpu.VMEM((128 + PLANE,), jnp.float32),
            pltpu.VMEM((128 + PLANE,), jnp.float32),
            pltpu.VMEM((128 + PLANE,), jnp.float32),
            pltpu.SemaphoreType.DMA,
            pltpu.SemaphoreType.DMA,
            pltpu.SemaphoreType.DMA,
            pltpu.SemaphoreType.DMA,
            pltpu.SemaphoreType.DMA,
        ],
    )(_sc_gridding_reverse)


def _sc_gridding_reverse(g_hbm, out_hbm, gb0, gb1, gb2,
                         ob0x, ob0y, ob0z, ob1x, ob1y, ob1z,
                         is0, is1, is2, os0, os1):
    gbufs = (gb0, gb1, gb2)
    obufs = ((ob0x, ob0y, ob0z), (ob1x, ob1y, ob1z))
    isems = (is0, is1, is2)
    osems = (os0, os1)

    wid = lax.axis_index("s") * NC + lax.axis_index("c")
    b = wid // 4
    z0 = (wid % 4) * ZPW

    iota_f = jnp.arange(L, dtype=jnp.int32).astype(jnp.float32)
    zeros_v = jnp.zeros((L,), jnp.float32)
    inv32 = jnp.float32(1.0 / 32.0)

    def load_plane(z, k):
        return pltpu.async_copy(
            g_hbm.at[b, z], gbufs[k].at[pl.ds(1, S), :], isems[k])

    in_dma = [load_plane(jnp.maximum(z0 - 1, 0), 0), load_plane(z0, 1)]
    out_dma = []

    for i in range(ZPW):
        z = z0 + i
        gz = gbufs[(i + 1) % 3]
        gzm1 = gbufs[i % 3]
        obx, oby, obz = obufs[i % 2]

        if i == 0:
            in_dma[0].wait()
        in_dma[1].wait()
        if i < ZPW - 1:
            in_dma = [None, load_plane(z + 1, (i + 2) % 3)]
        if i >= 2:
            for h in out_dma[i - 2]:
                h.wait()

        thr = jnp.where(z > 0, jnp.float32(0.0), jnp.float32(jnp.inf))
        zs = (z.astype(jnp.float32) - 32.0) * inv32

        def x_body(xi, _):
            x0 = xi * L
            xs = (x0.astype(jnp.float32) + iota_f - 32.0) * inv32
            thrv = jnp.where(xs > jnp.float32(-1.0), thr, jnp.float32(jnp.inf))
            xm1 = x0 - 1

            def stage_b(y, wsum, sx, sy, sz):
                r32 = inv32 / wsum
                ys = (y.astype(jnp.float32) - 33.0) * inv32
                cond = wsum > thrv
                px = jnp.where(cond, xs - sx * r32, zeros_v)
                py = jnp.where(cond, ys - sy * r32, zeros_v)
                pz = jnp.where(cond, zs - sz * r32, zeros_v)
                o = 64 + y * S + x0
                obx[pl.ds(o, L)] = px
                oby[pl.ds(o, L)] = py
                obz[pl.ds(o, L)] = pz

            def y_body(y, carry):
                p_prev, q_prev, pzm1_prev, wsum1, sx1, sy1, sz1 = carry
                y1 = y + 1
                a = gz[y1, pl.ds(x0, L)]
                ax = gz[y1, pl.ds(xm1, L)]
                c = gzm1[y1, pl.ds(x0, L)]
                cx = gzm1[y1, pl.ds(xm1, L)]
                pa_z = a + ax
                pa_zm1 = c + cx
                q = ax + cx
                p_cur = pa_z + pa_zm1
                wsum = p_cur + p_prev
                sx = q + q_prev
                sy = p_prev
                sz = pa_zm1 + pzm1_prev
                stage_b(y, wsum1, sx1, sy1, sz1)
                return (p_cur, q, pa_zm1, wsum, sx, sy, sz)

            carry = lax.fori_loop(0, S, y_body, (zeros_v,) * 7, unroll=8)
            stage_b(jnp.int32(S), carry[3], carry[4], carry[5], carry[6])
            return 0

        lax.fori_loop(0, S // L, x_body, 0)

        for k in range(S // L):
            obx[pl.ds(128 + k * L, L)] = zeros_v
            oby[pl.ds(128 + k * L, L)] = zeros_v
            obz[pl.ds(128 + k * L, L)] = zeros_v

        sem = osems[i % 2]
        out_dma.append([
            pltpu.async_copy(obx.at[pl.ds(128, PLANE)],
                             out_hbm.at[0, b, pl.ds(z * PLANE, PLANE)], sem),
            pltpu.async_copy(oby.at[pl.ds(128, PLANE)],
                             out_hbm.at[1, b, pl.ds(z * PLANE, PLANE)], sem),
            pltpu.async_copy(obz.at[pl.ds(128, PLANE)],
                             out_hbm.at[2, b, pl.ds(z * PLANE, PLANE)], sem),
        ])

    for hs in out_dma[-2:]:
        for h in hs:
            h.wait()


def kernel(grid):
    planar = _build()(grid)
    return planar.transpose(1, 2, 0)


# speedup vs baseline: 1.0060x; 1.0060x over previous
import functools

import jax
import jax.numpy as jnp
from jax import lax
from jax.experimental import pallas as pl
from jax.experimental.pallas import tpu as pltpu
from jax.experimental.pallas import tpu_sc as plsc

S = 64
PLANE = S * S
NC, NS, L = 2, 16, 16
B = 8
ZPW = 16
N_PTS = S * S * S


@functools.lru_cache(maxsize=None)
def _build():
    mesh = plsc.VectorSubcoreMesh(core_axis_name="c", subcore_axis_name="s",
                                  num_cores=NC, num_subcores=NS)
    return functools.partial(
        pl.kernel,
        out_type=jax.ShapeDtypeStruct((3, B, N_PTS), jnp.float32),
        mesh=mesh,
        compiler_params=pltpu.CompilerParams(needs_layout_passes=False),
        scratch_types=[
            pltpu.VMEM((S + 1, S), jnp.float32),
            pltpu.VMEM((S + 1, S), jnp.float32),
            pltpu.VMEM((S + 1, S), jnp.float32),
            pltpu.VMEM((128 + PLANE,), jnp.float32),
            pltpu.VMEM((128 + PLANE,), jnp.float32),
            pltpu.VMEM((128 + PLANE,), jnp.float32),
            pltpu.VMEM((128 + PLANE,), jnp.float32),
            pltpu.VMEM((128 + PLANE,), jnp.float32),
            pltpu.VMEM((128 + PLANE,), jnp.float32),
            pltpu.SemaphoreType.DMA,
            pltpu.SemaphoreType.DMA,
            pltpu.SemaphoreType.DMA,
            pltpu.SemaphoreType.DMA,
            pltpu.SemaphoreType.DMA,
        ],
    )(_sc_gridding_reverse)


def _sc_gridding_reverse(g_hbm, out_hbm, gb0, gb1, gb2,
                         ob0x, ob0y, ob0z, ob1x, ob1y, ob1z,
                         is0, is1, is2, os0, os1):
    gbufs = (gb0, gb1, gb2)
    obufs = ((ob0x, ob0y, ob0z), (ob1x, ob1y, ob1z))
    isems = (is0, is1, is2)
    osems = (os0, os1)

    wid = lax.axis_index("s") * NC + lax.axis_index("c")
    b = wid // 4
    z0 = (wid % 4) * ZPW

    iota_f = jnp.arange(L, dtype=jnp.int32).astype(jnp.float32)
    zeros_v = jnp.zeros((L,), jnp.float32)
    inv32 = jnp.float32(1.0 / 32.0)

    def load_plane(z, k):
        return pltpu.async_copy(
            g_hbm.at[b, z], gbufs[k].at[pl.ds(1, S), :], isems[k])

    in_dma = [load_plane(jnp.maximum(z0 - 1, 0), 0), load_plane(z0, 1)]
    out_dma = []

    for i in range(ZPW):
        z = z0 + i
        gz = gbufs[(i + 1) % 3]
        gzm1 = gbufs[i % 3]
        obx, oby, obz = obufs[i % 2]

        if i == 0:
            in_dma[0].wait()
        in_dma[1].wait()
        if i < ZPW - 1:
            in_dma = [None, load_plane(z + 1, (i + 2) % 3)]
        if i >= 2:
            for h in out_dma[i - 2]:
                h.wait()

        thr = jnp.where(z > 0, jnp.float32(0.0), jnp.float32(jnp.inf))
        zs = (z.astype(jnp.float32) - 32.0) * inv32

        def x_body(xi, _):
            x0 = xi * L
            xs = (x0.astype(jnp.float32) + iota_f - 32.0) * inv32
            thrv = jnp.where(xs > jnp.float32(-1.0), thr, jnp.float32(jnp.inf))
            xm1 = x0 - 1

            def stage_b(y, wsum, sx, sy, sz):
                r32 = inv32 / wsum
                ys = (y.astype(jnp.float32) - 33.0) * inv32
                cond = wsum > thrv
                px = jnp.where(cond, xs - sx * r32, zeros_v)
                py = jnp.where(cond, ys - sy * r32, zeros_v)
                pz = jnp.where(cond, zs - sz * r32, zeros_v)
                o = 64 + y * S + x0
                obx[pl.ds(o, L)] = px
                oby[pl.ds(o, L)] = py
                obz[pl.ds(o, L)] = pz

            def y_body(y, carry):
                p_prev, q_prev, pzm1_prev, wsum1, sx1, sy1, sz1 = carry
                y1 = y + 1
                a = gz[y1, pl.ds(x0, L)]
                ax = gz[y1, pl.ds(xm1, L)]
                c = gzm1[y1, pl.ds(x0, L)]
                cx = gzm1[y1, pl.ds(xm1, L)]
                pa_z = a + ax
                pa_zm1 = c + cx
                q = ax + cx
                p_cur = pa_z + pa_zm1
                wsum = p_cur + p_prev
                sx = q + q_prev
                sy = p_prev
                sz = pa_zm1 + pzm1_prev
                stage_b(y, wsum1, sx1, sy1, sz1)
                return (p_cur, q, pa_zm1, wsum, sx, sy, sz)

            carry = lax.fori_loop(0, S, y_body, (zeros_v,) * 7, unroll=4)
            stage_b(jnp.int32(S), carry[3], carry[4], carry[5], carry[6])
            return 0

        lax.fori_loop(0, S // L, x_body, 0)

        for k in range(S // L):
            obx[pl.ds(128 + k * L, L)] = zeros_v
            oby[pl.ds(128 + k * L, L)] = zeros_v
            obz[pl.ds(128 + k * L, L)] = zeros_v

        sem = osems[i % 2]
        out_dma.append([
            pltpu.async_copy(obx.at[pl.ds(128, PLANE)],
                             out_hbm.at[0, b, pl.ds(z * PLANE, PLANE)], sem),
            pltpu.async_copy(oby.at[pl.ds(128, PLANE)],
                             out_hbm.at[1, b, pl.ds(z * PLANE, PLANE)], sem),
            pltpu.async_copy(obz.at[pl.ds(128, PLANE)],
                             out_hbm.at[2, b, pl.ds(z * PLANE, PLANE)], sem),
        ])

    for hs in out_dma[-2:]:
        for h in hs:
            h.wait()


def kernel(grid):
    planar = _build()(grid)
    return planar.transpose(1, 2, 0)


# speedup vs baseline: 1.2725x; 1.2649x over previous
import functools

import jax
import jax.numpy as jnp
from jax import lax
from jax.experimental import pallas as pl
from jax.experimental.pallas import tpu as pltpu
from jax.experimental.pallas import tpu_sc as plsc

S = 64
PLANE = S * S
NC, NS, L = 2, 16, 16
B = 8
ZPW = 16
N_PTS = S * S * S


@functools.lru_cache(maxsize=None)
def _build():
    mesh = plsc.VectorSubcoreMesh(core_axis_name="c", subcore_axis_name="s",
                                  num_cores=NC, num_subcores=NS)
    return functools.partial(
        pl.kernel,
        out_type=jax.ShapeDtypeStruct((3, B, N_PTS), jnp.float32),
        mesh=mesh,
        compiler_params=pltpu.CompilerParams(needs_layout_passes=False),
        scratch_types=[
            pltpu.VMEM((S + 1, S), jnp.float32),
            pltpu.VMEM((S + 1, S), jnp.float32),
            pltpu.VMEM((S + 1, S), jnp.float32),
            pltpu.VMEM((128 + PLANE,), jnp.float32),
            pltpu.VMEM((128 + PLANE,), jnp.float32),
            pltpu.VMEM((128 + PLANE,), jnp.float32),
            pltpu.VMEM((128 + PLANE,), jnp.float32),
            pltpu.VMEM((128 + PLANE,), jnp.float32),
            pltpu.VMEM((128 + PLANE,), jnp.float32),
            pltpu.SemaphoreType.DMA,
            pltpu.SemaphoreType.DMA,
            pltpu.SemaphoreType.DMA,
            pltpu.SemaphoreType.DMA,
            pltpu.SemaphoreType.DMA,
        ],
    )(_sc_gridding_reverse)


def _sc_gridding_reverse(g_hbm, out_hbm, gb0, gb1, gb2,
                         ob0x, ob0y, ob0z, ob1x, ob1y, ob1z,
                         is0, is1, is2, os0, os1):
    gbufs = (gb0, gb1, gb2)
    obufs = ((ob0x, ob0y, ob0z), (ob1x, ob1y, ob1z))
    isems = (is0, is1, is2)
    osems = (os0, os1)

    wid = lax.axis_index("s") * NC + lax.axis_index("c")
    b = wid // 4
    z0 = (wid % 4) * ZPW

    iota_f = jnp.arange(L, dtype=jnp.int32).astype(jnp.float32)
    zeros_v = jnp.zeros((L,), jnp.float32)
    inv32 = jnp.float32(1.0 / 32.0)

    def load_plane(z, k):
        return pltpu.async_copy(
            g_hbm.at[b, z], gbufs[k].at[pl.ds(1, S), :], isems[k])

    in_dma = [load_plane(jnp.maximum(z0 - 1, 0), 0), load_plane(z0, 1)]
    out_dma = []

    for i in range(ZPW):
        z = z0 + i
        gz = gbufs[(i + 1) % 3]
        gzm1 = gbufs[i % 3]
        obx, oby, obz = obufs[i % 2]

        if i == 0:
            in_dma[0].wait()
        in_dma[1].wait()
        if i < ZPW - 1:
            in_dma = [None, load_plane(z + 1, (i + 2) % 3)]
        if i >= 2:
            for h in out_dma[i - 2]:
                h.wait()

        thr = jnp.where(z > 0, jnp.float32(0.0), jnp.float32(jnp.inf))
        zs = (z.astype(jnp.float32) - 32.0) * inv32

        def x_body(xi, _):
            x0s = (xi * (2 * L), xi * (2 * L) + L)
            xss = tuple((x0.astype(jnp.float32) + iota_f - 32.0) * inv32
                        for x0 in x0s)
            thrvs = tuple(
                jnp.where(xs > jnp.float32(-1.0), thr, jnp.float32(jnp.inf))
                for xs in xss)

            def stage_b(k, y, wsum, sx, sy, sz):
                r32 = inv32 / wsum
                ys = (y.astype(jnp.float32) - 33.0) * inv32
                cond = wsum > thrvs[k]
                px = jnp.where(cond, xss[k] - sx * r32, zeros_v)
                py = jnp.where(cond, ys - sy * r32, zeros_v)
                pz = jnp.where(cond, zs - sz * r32, zeros_v)
                o = 64 + y * S + x0s[k]
                obx[pl.ds(o, L)] = px
                oby[pl.ds(o, L)] = py
                obz[pl.ds(o, L)] = pz

            def stage_a(k, y1, carry3):
                p_prev, q_prev, pzm1_prev = carry3
                x0 = x0s[k]
                a = gz[y1, pl.ds(x0, L)]
                ax = gz[y1, pl.ds(x0 - 1, L)]
                c = gzm1[y1, pl.ds(x0, L)]
                cx = gzm1[y1, pl.ds(x0 - 1, L)]
                pa_z = a + ax
                pa_zm1 = c + cx
                q = ax + cx
                p_cur = pa_z + pa_zm1
                wsum = p_cur + p_prev
                sx = q + q_prev
                sy = p_prev
                sz = pa_zm1 + pzm1_prev
                return (p_cur, q, pa_zm1), (wsum, sx, sy, sz)

            def y_body(y, carry):
                ra0, b0, ra1, b1 = carry
                y1 = y + 1
                ra0n, b0n = stage_a(0, y1, ra0)
                ra1n, b1n = stage_a(1, y1, ra1)
                stage_b(0, y, *b0)
                stage_b(1, y, *b1)
                return (ra0n, b0n, ra1n, b1n)

            z3 = (zeros_v,) * 3
            z4 = (zeros_v,) * 4
            carry = lax.fori_loop(0, S, y_body, (z3, z4, z3, z4), unroll=4)
            stage_b(0, jnp.int32(S), *carry[1])
            stage_b(1, jnp.int32(S), *carry[3])
            return 0

        lax.fori_loop(0, S // (2 * L), x_body, 0)

        for k in range(S // L):
            obx[pl.ds(128 + k * L, L)] = zeros_v
            oby[pl.ds(128 + k * L, L)] = zeros_v
            obz[pl.ds(128 + k * L, L)] = zeros_v

        sem = osems[i % 2]
        out_dma.append([
            pltpu.async_copy(obx.at[pl.ds(128, PLANE)],
                             out_hbm.at[0, b, pl.ds(z * PLANE, PLANE)], sem),
            pltpu.async_copy(oby.at[pl.ds(128, PLANE)],
                             out_hbm.at[1, b, pl.ds(z * PLANE, PLANE)], sem),
            pltpu.async_copy(obz.at[pl.ds(128, PLANE)],
                             out_hbm.at[2, b, pl.ds(z * PLANE, PLANE)], sem),
        ])

    for hs in out_dma[-2:]:
        for h in hs:
            h.wait()


def kernel(grid):
    planar = _build()(grid)
    return planar.transpose(1, 2, 0)


# speedup vs baseline: 1.4389x; 1.1307x over previous
import functools

import jax
import jax.numpy as jnp
from jax import lax
from jax.experimental import pallas as pl
from jax.experimental.pallas import tpu as pltpu
from jax.experimental.pallas import tpu_sc as plsc

S = 64
PLANE = S * S
NC, NS, L = 2, 16, 16
B = 8
ZPW = 16
N_PTS = S * S * S


@functools.lru_cache(maxsize=None)
def _build():
    mesh = plsc.VectorSubcoreMesh(core_axis_name="c", subcore_axis_name="s",
                                  num_cores=NC, num_subcores=NS)
    return functools.partial(
        pl.kernel,
        out_type=jax.ShapeDtypeStruct((3, B, N_PTS), jnp.float32),
        mesh=mesh,
        compiler_params=pltpu.CompilerParams(needs_layout_passes=False),
        scratch_types=[
            pltpu.VMEM((S + 1, S), jnp.float32),
            pltpu.VMEM((S + 1, S), jnp.float32),
            pltpu.VMEM((S + 1, S), jnp.float32),
            pltpu.VMEM((128 + PLANE,), jnp.float32),
            pltpu.VMEM((128 + PLANE,), jnp.float32),
            pltpu.VMEM((128 + PLANE,), jnp.float32),
            pltpu.VMEM((128 + PLANE,), jnp.float32),
            pltpu.VMEM((128 + PLANE,), jnp.float32),
            pltpu.VMEM((128 + PLANE,), jnp.float32),
            pltpu.SemaphoreType.DMA,
            pltpu.SemaphoreType.DMA,
            pltpu.SemaphoreType.DMA,
            pltpu.SemaphoreType.DMA,
            pltpu.SemaphoreType.DMA,
        ],
    )(_sc_gridding_reverse)


def _sc_gridding_reverse(g_hbm, out_hbm, gb0, gb1, gb2,
                         ob0x, ob0y, ob0z, ob1x, ob1y, ob1z,
                         is0, is1, is2, os0, os1):
    gbufs = (gb0, gb1, gb2)
    obufs = ((ob0x, ob0y, ob0z), (ob1x, ob1y, ob1z))
    isems = (is0, is1, is2)
    osems = (os0, os1)

    wid = lax.axis_index("s") * NC + lax.axis_index("c")
    b = wid // 4
    z0 = (wid % 4) * ZPW

    iota_f = jnp.arange(L, dtype=jnp.int32).astype(jnp.float32)
    zeros_v = jnp.zeros((L,), jnp.float32)
    inv32 = jnp.float32(1.0 / 32.0)

    def load_plane(z, k):
        return pltpu.async_copy(
            g_hbm.at[b, z], gbufs[k].at[pl.ds(1, S), :], isems[k])

    in_dma = [load_plane(jnp.maximum(z0 - 1, 0), 0), load_plane(z0, 1)]
    out_dma = []

    for i in range(ZPW):
        z = z0 + i
        gz = gbufs[(i + 1) % 3]
        gzm1 = gbufs[i % 3]
        obx, oby, obz = obufs[i % 2]

        if i == 0:
            in_dma[0].wait()
        in_dma[1].wait()
        if i < ZPW - 1:
            in_dma = [None, load_plane(z + 1, (i + 2) % 3)]
        if i >= 2:
            for h in out_dma[i - 2]:
                h.wait()

        thr = jnp.where(z > 0, jnp.float32(0.0), jnp.float32(jnp.inf))
        zs = (z.astype(jnp.float32) - 32.0) * inv32

        def x_body(xi, _):
            x0s = (xi * (4 * L), xi * (4 * L) + L, xi * (4 * L) + 2 * L, xi * (4 * L) + 3 * L)
            xss = tuple((x0.astype(jnp.float32) + iota_f - 32.0) * inv32
                        for x0 in x0s)
            thrvs = tuple(
                jnp.where(xs > jnp.float32(-1.0), thr, jnp.float32(jnp.inf))
                for xs in xss)

            def stage_b(k, y, wsum, sx, sy, sz):
                r32 = inv32 / wsum
                ys = (y.astype(jnp.float32) - 33.0) * inv32
                cond = wsum > thrvs[k]
                px = jnp.where(cond, xss[k] - sx * r32, zeros_v)
                py = jnp.where(cond, ys - sy * r32, zeros_v)
                pz = jnp.where(cond, zs - sz * r32, zeros_v)
                o = 64 + y * S + x0s[k]
                obx[pl.ds(o, L)] = px
                oby[pl.ds(o, L)] = py
                obz[pl.ds(o, L)] = pz

            def stage_a(k, y1, carry3):
                p_prev, q_prev, pzm1_prev = carry3
                x0 = x0s[k]
                a = gz[y1, pl.ds(x0, L)]
                ax = gz[y1, pl.ds(x0 - 1, L)]
                c = gzm1[y1, pl.ds(x0, L)]
                cx = gzm1[y1, pl.ds(x0 - 1, L)]
                pa_z = a + ax
                pa_zm1 = c + cx
                q = ax + cx
                p_cur = pa_z + pa_zm1
                wsum = p_cur + p_prev
                sx = q + q_prev
                sy = p_prev
                sz = pa_zm1 + pzm1_prev
                return (p_cur, q, pa_zm1), (wsum, sx, sy, sz)

            def y_body(y, carry):
                ras, bs = carry
                y1 = y + 1
                new = [stage_a(k, y1, ras[k]) for k in range(4)]
                for k in range(4):
                    stage_b(k, y, *bs[k])
                return (tuple(n[0] for n in new), tuple(n[1] for n in new))

            z3 = (zeros_v,) * 3
            z4 = (zeros_v,) * 4
            carry = lax.fori_loop(0, S, y_body,
                                  ((z3,) * 4, (z4,) * 4), unroll=2)
            for k in range(4):
                stage_b(k, jnp.int32(S), *carry[1][k])
            return 0

        lax.fori_loop(0, S // (4 * L), x_body, 0)

        for k in range(S // L):
            obx[pl.ds(128 + k * L, L)] = zeros_v
            oby[pl.ds(128 + k * L, L)] = zeros_v
            obz[pl.ds(128 + k * L, L)] = zeros_v

        sem = osems[i % 2]
        out_dma.append([
            pltpu.async_copy(obx.at[pl.ds(128, PLANE)],
                             out_hbm.at[0, b, pl.ds(z * PLANE, PLANE)], sem),
            pltpu.async_copy(oby.at[pl.ds(128, PLANE)],
                             out_hbm.at[1, b, pl.ds(z * PLANE, PLANE)], sem),
            pltpu.async_copy(obz.at[pl.ds(128, PLANE)],
                             out_hbm.at[2, b, pl.ds(z * PLANE, PLANE)], sem),
        ])

    for hs in out_dma[-2:]:
        for h in hs:
            h.wait()


def kernel(grid):
    planar = _build()(grid)
    return planar.transpose(1, 2, 0)
